# Initial kernel scaffold; baseline (speedup 1.0000x reference)
import jax
import jax.numpy as jnp
from jax.experimental import pallas as pl


def kernel(emb_F, emb_S, edge_index):
    raise NotImplementedError("write your pallas kernel here")


# speedup vs baseline: 1.0046x; 1.0046x over previous
import jax
import jax.numpy as jnp
from jax.experimental import pallas as pl
from jax.experimental.pallas import tpu as pltpu

_FSIM_WEIGHT = 0.5
_EPS = 1e-8


def _sim_mask_body(qf, kf, qs, ks, qfn, kfn, qsn, ksn, em, s_out, r_out):
    bq = s_out.shape[0]
    bn = s_out.shape[1]
    dotf = jax.lax.dot_general(qf[...], kf[...], (((1,), (1,)), ((), ())),
                               preferred_element_type=jnp.float32)
    dots = jax.lax.dot_general(qs[...], ks[...], (((1,), (1,)), ((), ())),
                               preferred_element_type=jnp.float32)
    f_sim = dotf / jnp.maximum(qfn[...] * kfn[...], _EPS)
    s_sim = dots / jnp.maximum(qsn[...] * ksn[...], _EPS)
    sim = _FSIM_WEIGHT * f_sim + (1.0 - _FSIM_WEIGHT) * s_sim

    bi = pl.program_id(0)
    bj = pl.program_id(1)
    gr = bi * bq + jax.lax.broadcasted_iota(jnp.int32, (bq, bn), 0)
    gc = bj * bn + jax.lax.broadcasted_iota(jnp.int32, (bq, bn), 1)
    dm = (gr == gc).astype(jnp.float32)
    emf = em[...].astype(jnp.float32)
    s_out[...] = sim + (emf + dm) * -99.0
    r_out[...] = sim + (1.0 - emf) * 99.0


def _masked_sims(emb_F, emb_S, edge_mask_u8, bq=256, bn=2048):
    q = edge_mask_u8.shape[0]
    n, d = emb_F.shape
    grid = (q // bq, n // bn)

    qfn = jnp.linalg.norm(emb_F[:q], axis=1, keepdims=True)
    kfn = jnp.linalg.norm(emb_F, axis=1, keepdims=True).reshape(1, n)
    qsn = jnp.linalg.norm(emb_S[:q], axis=1, keepdims=True)
    ksn = jnp.linalg.norm(emb_S, axis=1, keepdims=True).reshape(1, n)

    return pl.pallas_call(
        _sim_mask_body,
        grid=grid,
        in_specs=[
            pl.BlockSpec((bq, d), lambda i, j: (i, 0)),
            pl.BlockSpec((bn, d), lambda i, j: (j, 0)),
            pl.BlockSpec((bq, d), lambda i, j: (i, 0)),
            pl.BlockSpec((bn, d), lambda i, j: (j, 0)),
            pl.BlockSpec((bq, 1), lambda i, j: (i, 0)),
            pl.BlockSpec((1, bn), lambda i, j: (0, j)),
            pl.BlockSpec((bq, 1), lambda i, j: (i, 0)),
            pl.BlockSpec((1, bn), lambda i, j: (0, j)),
            pl.BlockSpec((bq, bn), lambda i, j: (i, j)),
        ],
        out_specs=[
            pl.BlockSpec((bq, bn), lambda i, j: (i, j)),
            pl.BlockSpec((bq, bn), lambda i, j: (i, j)),
        ],
        out_shape=[
            jax.ShapeDtypeStruct((q, n), jnp.float32),
            jax.ShapeDtypeStruct((q, n), jnp.float32),
        ],
    )(emb_F[:q], emb_F, emb_S[:q], emb_S, qfn, kfn, qsn, ksn, edge_mask_u8)


def kernel(emb_F, emb_S, edge_index):
    n, d = emb_F.shape
    q = n // 8
    add_num = 1024
    rm_num = 1024

    row = edge_index[0]
    col = edge_index[1]
    row_safe = jnp.where(row < q, row, q)
    edge_mask_u8 = jnp.zeros((q, n), dtype=jnp.uint8).at[row_safe, col].set(
        1, mode='drop')

    s_arr, r_arr = _masked_sims(emb_F, emb_S, edge_mask_u8)

    add_vals, add_idx = jax.lax.top_k(s_arr.reshape(-1), add_num)
    neg_vals, rm_idx = jax.lax.top_k(-r_arr.reshape(-1), rm_num)
    rm_vals = -neg_vals
    add_inds = jnp.stack([add_idx // n, add_idx % n], axis=1)
    rm_inds = jnp.stack([rm_idx // n, rm_idx % n], axis=1)
    return add_vals, add_inds, rm_vals, rm_inds


# speedup vs baseline: 38.5458x; 38.3706x over previous
import jax
import jax.numpy as jnp
from jax.experimental import pallas as pl
from jax.experimental.pallas import tpu as pltpu

_FSIM_WEIGHT = 0.5
_EPS = 1e-8
_W = 256


def _sim_mask_body(qf, kf, qs, ks, qfn, kfn, qsn, ksn, em, s_out, r_out):
    bq = s_out.shape[0]
    bn = s_out.shape[1]
    dotf = jax.lax.dot_general(qf[...], kf[...], (((1,), (1,)), ((), ())),
                               preferred_element_type=jnp.float32)
    dots = jax.lax.dot_general(qs[...], ks[...], (((1,), (1,)), ((), ())),
                               preferred_element_type=jnp.float32)
    f_sim = dotf / jnp.maximum(qfn[...] * kfn[...], _EPS)
    s_sim = dots / jnp.maximum(qsn[...] * ksn[...], _EPS)
    sim = _FSIM_WEIGHT * f_sim + (1.0 - _FSIM_WEIGHT) * s_sim

    bi = pl.program_id(0)
    bj = pl.program_id(1)
    gr = bi * bq + jax.lax.broadcasted_iota(jnp.int32, (bq, bn), 0)
    gc = bj * bn + jax.lax.broadcasted_iota(jnp.int32, (bq, bn), 1)
    dm = (gr == gc).astype(jnp.float32)
    emf = em[...].astype(jnp.float32)
    s_out[...] = sim + (emf + dm) * -99.0
    r_out[...] = sim + (1.0 - emf) * 99.0


def _pool_body(s, r, sp_out, rp_out):
    n = s.shape[1]
    for w in range(n // _W):
        sl = slice(w * _W, (w + 1) * _W)
        sp_out[:, w:w + 1] = jnp.max(s[:, sl], axis=1, keepdims=True)
        rp_out[:, w:w + 1] = jnp.min(r[:, sl], axis=1, keepdims=True)


def _pooled_extrema(s_arr, r_arr, bq=64):
    q, n = s_arr.shape
    nw = n // _W
    return pl.pallas_call(
        _pool_body,
        grid=(q // bq,),
        in_specs=[
            pl.BlockSpec((bq, n), lambda i: (i, 0)),
            pl.BlockSpec((bq, n), lambda i: (i, 0)),
        ],
        out_specs=[
            pl.BlockSpec((bq, nw), lambda i: (i, 0)),
            pl.BlockSpec((bq, nw), lambda i: (i, 0)),
        ],
        out_shape=[
            jax.ShapeDtypeStruct((q, nw), jnp.float32),
            jax.ShapeDtypeStruct((q, nw), jnp.float32),
        ],
    )(s_arr, r_arr)


def _masked_sims(emb_F, emb_S, edge_mask_u8, bq=256, bn=2048):
    q = edge_mask_u8.shape[0]
    n, d = emb_F.shape
    grid = (q // bq, n // bn)

    qfn = jnp.linalg.norm(emb_F[:q], axis=1, keepdims=True)
    kfn = jnp.linalg.norm(emb_F, axis=1, keepdims=True).reshape(1, n)
    qsn = jnp.linalg.norm(emb_S[:q], axis=1, keepdims=True)
    ksn = jnp.linalg.norm(emb_S, axis=1, keepdims=True).reshape(1, n)

    return pl.pallas_call(
        _sim_mask_body,
        grid=grid,
        in_specs=[
            pl.BlockSpec((bq, d), lambda i, j: (i, 0)),
            pl.BlockSpec((bn, d), lambda i, j: (j, 0)),
            pl.BlockSpec((bq, d), lambda i, j: (i, 0)),
            pl.BlockSpec((bn, d), lambda i, j: (j, 0)),
            pl.BlockSpec((bq, 1), lambda i, j: (i, 0)),
            pl.BlockSpec((1, bn), lambda i, j: (0, j)),
            pl.BlockSpec((bq, 1), lambda i, j: (i, 0)),
            pl.BlockSpec((1, bn), lambda i, j: (0, j)),
            pl.BlockSpec((bq, bn), lambda i, j: (i, j)),
        ],
        out_specs=[
            pl.BlockSpec((bq, bn), lambda i, j: (i, j)),
            pl.BlockSpec((bq, bn), lambda i, j: (i, j)),
        ],
        out_shape=[
            jax.ShapeDtypeStruct((q, n), jnp.float32),
            jax.ShapeDtypeStruct((q, n), jnp.float32),
        ],
    )(emb_F[:q], emb_F, emb_S[:q], emb_S, qfn, kfn, qsn, ksn, edge_mask_u8)


def _windowed_topk(flat, pooled_flat, k, n):
    _, win_idx = jax.lax.top_k(pooled_flat, k)
    win_sorted = jnp.sort(win_idx)
    gidx = win_sorted[:, None] * _W + jnp.arange(_W, dtype=jnp.int32)[None, :]
    cand = jnp.take(flat, gidx.reshape(-1), axis=0)
    vals, ci = jax.lax.top_k(cand, k)
    flat_idx = gidx.reshape(-1)[ci]
    inds = jnp.stack([flat_idx // n, flat_idx % n], axis=1)
    return vals, inds


def kernel(emb_F, emb_S, edge_index):
    n, d = emb_F.shape
    q = n // 8
    add_num = 1024
    rm_num = 1024

    row = edge_index[0]
    col = edge_index[1]
    row_safe = jnp.where(row < q, row, q)
    edge_mask_u8 = jnp.zeros((q, n), dtype=jnp.uint8).at[row_safe, col].set(
        1, mode='drop')

    s_arr, r_arr = _masked_sims(emb_F, emb_S, edge_mask_u8)
    s_pool, r_pool = _pooled_extrema(s_arr, r_arr)

    add_vals, add_inds = _windowed_topk(
        s_arr.reshape(-1), s_pool.reshape(-1), add_num, n)
    neg_vals, rm_inds = _windowed_topk(
        -r_arr.reshape(-1), -r_pool.reshape(-1), rm_num, n)
    rm_vals = -neg_vals
    return add_vals, add_inds, rm_vals, rm_inds


# speedup vs baseline: 69.0897x; 1.7924x over previous
import jax
import jax.numpy as jnp
from jax.experimental import pallas as pl
from jax.experimental.pallas import tpu as pltpu

_FSIM_WEIGHT = 0.5
_EPS = 1e-8
_W = 256


def _sim_mask_body(qf, kf, qs, ks, qfn, kfn, qsn, ksn, em, s_out, r_out):
    bq = s_out.shape[0]
    bn = s_out.shape[1]
    dotf = jax.lax.dot_general(qf[...], kf[...], (((1,), (1,)), ((), ())),
                               preferred_element_type=jnp.float32)
    dots = jax.lax.dot_general(qs[...], ks[...], (((1,), (1,)), ((), ())),
                               preferred_element_type=jnp.float32)
    f_sim = dotf / jnp.maximum(qfn[...] * kfn[...], _EPS)
    s_sim = dots / jnp.maximum(qsn[...] * ksn[...], _EPS)
    sim = _FSIM_WEIGHT * f_sim + (1.0 - _FSIM_WEIGHT) * s_sim

    bi = pl.program_id(0)
    bj = pl.program_id(1)
    gr = bi * bq + jax.lax.broadcasted_iota(jnp.int32, (bq, bn), 0)
    gc = bj * bn + jax.lax.broadcasted_iota(jnp.int32, (bq, bn), 1)
    dm = (gr == gc).astype(jnp.float32)
    emf = (em[...] > 0.0).astype(jnp.float32)
    s_out[...] = sim + (emf + dm) * -99.0
    r_out[...] = sim + (1.0 - emf) * 99.0


def _pool_body(s, r, sp_out, rp_out):
    n = s.shape[1]
    for w in range(n // _W):
        sl = slice(w * _W, (w + 1) * _W)
        sp_out[:, w:w + 1] = jnp.max(s[:, sl], axis=1, keepdims=True)
        rp_out[:, w:w + 1] = jnp.min(r[:, sl], axis=1, keepdims=True)


def _pooled_extrema(s_arr, r_arr, bq=64):
    q, n = s_arr.shape
    nw = n // _W
    return pl.pallas_call(
        _pool_body,
        grid=(q // bq,),
        in_specs=[
            pl.BlockSpec((bq, n), lambda i: (i, 0)),
            pl.BlockSpec((bq, n), lambda i: (i, 0)),
        ],
        out_specs=[
            pl.BlockSpec((bq, nw), lambda i: (i, 0)),
            pl.BlockSpec((bq, nw), lambda i: (i, 0)),
        ],
        out_shape=[
            jax.ShapeDtypeStruct((q, nw), jnp.float32),
            jax.ShapeDtypeStruct((q, nw), jnp.float32),
        ],
    )(s_arr, r_arr)


def _masked_sims(emb_F, emb_S, edge_mask_u8, bq=256, bn=2048):
    q = edge_mask_u8.shape[0]
    n, d = emb_F.shape
    grid = (q // bq, n // bn)

    qfn = jnp.linalg.norm(emb_F[:q], axis=1, keepdims=True)
    kfn = jnp.linalg.norm(emb_F, axis=1, keepdims=True).reshape(1, n)
    qsn = jnp.linalg.norm(emb_S[:q], axis=1, keepdims=True)
    ksn = jnp.linalg.norm(emb_S, axis=1, keepdims=True).reshape(1, n)

    return pl.pallas_call(
        _sim_mask_body,
        grid=grid,
        in_specs=[
            pl.BlockSpec((bq, d), lambda i, j: (i, 0)),
            pl.BlockSpec((bn, d), lambda i, j: (j, 0)),
            pl.BlockSpec((bq, d), lambda i, j: (i, 0)),
            pl.BlockSpec((bn, d), lambda i, j: (j, 0)),
            pl.BlockSpec((bq, 1), lambda i, j: (i, 0)),
            pl.BlockSpec((1, bn), lambda i, j: (0, j)),
            pl.BlockSpec((bq, 1), lambda i, j: (i, 0)),
            pl.BlockSpec((1, bn), lambda i, j: (0, j)),
            pl.BlockSpec((bq, bn), lambda i, j: (i, j)),
        ],
        out_specs=[
            pl.BlockSpec((bq, bn), lambda i, j: (i, j)),
            pl.BlockSpec((bq, bn), lambda i, j: (i, j)),
        ],
        out_shape=[
            jax.ShapeDtypeStruct((q, n), jnp.float32),
            jax.ShapeDtypeStruct((q, n), jnp.float32),
        ],
    )(emb_F[:q], emb_F, emb_S[:q], emb_S, qfn, kfn, qsn, ksn, edge_mask_u8)


def _windowed_topk(flat, pooled_flat, k, n, neg=False):
    _, win_idx = jax.lax.top_k(pooled_flat, k)
    win_sorted = jnp.sort(win_idx)
    gidx = win_sorted[:, None] * _W + jnp.arange(_W, dtype=jnp.int32)[None, :]
    cand = jnp.take(flat, gidx.reshape(-1), axis=0)
    if neg:
        cand = -cand
    vals, ci = jax.lax.top_k(cand, k)
    flat_idx = gidx.reshape(-1)[ci]
    inds = jnp.stack([flat_idx // n, flat_idx % n], axis=1)
    return vals, inds


def kernel(emb_F, emb_S, edge_index):
    n, d = emb_F.shape
    q = n // 8
    add_num = 1024
    rm_num = 1024

    row = edge_index[0]
    col = edge_index[1]
    flat_key = jnp.where(row < q, row * n + col, q * n)
    edge_cnt = jnp.zeros((q * n,), dtype=jnp.float32).at[flat_key].add(
        1.0, mode='drop').reshape(q, n)

    s_arr, r_arr = _masked_sims(emb_F, emb_S, edge_cnt)
    s_pool, r_pool = _pooled_extrema(s_arr, r_arr)

    add_vals, add_inds = _windowed_topk(
        s_arr.reshape(-1), s_pool.reshape(-1), add_num, n)
    neg_vals, rm_inds = _windowed_topk(
        r_arr.reshape(-1), -r_pool.reshape(-1), rm_num, n, neg=True)
    rm_vals = -neg_vals
    return add_vals, add_inds, rm_vals, rm_inds


# speedup vs baseline: 72.9074x; 1.0553x over previous
import jax
import jax.numpy as jnp
from jax.experimental import pallas as pl
from jax.experimental.pallas import tpu as pltpu

_FSIM_WEIGHT = 0.5
_EPS = 1e-8
_W = 256


def _sim_mask_body(qf, kf, qs, ks, qfn, kfn, qsn, ksn, em,
                   s_out, r_out, spt_out, rpt_out):
    bq = s_out.shape[0]
    bn = s_out.shape[1]
    dotf = jax.lax.dot_general(qf[...], kf[...], (((1,), (1,)), ((), ())),
                               preferred_element_type=jnp.float32)
    dots = jax.lax.dot_general(qs[...], ks[...], (((1,), (1,)), ((), ())),
                               preferred_element_type=jnp.float32)
    f_sim = dotf / jnp.maximum(qfn[...] * kfn[...], _EPS)
    s_sim = dots / jnp.maximum(qsn[...] * ksn[...], _EPS)
    sim = _FSIM_WEIGHT * f_sim + (1.0 - _FSIM_WEIGHT) * s_sim

    bi = pl.program_id(0)
    bj = pl.program_id(1)
    gr = bi * bq + jax.lax.broadcasted_iota(jnp.int32, (bq, bn), 0)
    gc = bj * bn + jax.lax.broadcasted_iota(jnp.int32, (bq, bn), 1)
    dm = (gr == gc).astype(jnp.float32)
    emf = (em[...] > 0.0).astype(jnp.float32)
    s_val = sim + (emf + dm) * -99.0
    r_val = sim + (1.0 - emf) * 99.0
    s_out[...] = s_val
    r_out[...] = r_val

    sp = jnp.concatenate(
        [jnp.max(s_val[:, w * _W:(w + 1) * _W], axis=1, keepdims=True)
         for w in range(bn // _W)], axis=1)
    rp = jnp.concatenate(
        [jnp.min(r_val[:, w * _W:(w + 1) * _W], axis=1, keepdims=True)
         for w in range(bn // _W)], axis=1)
    spt_out[...] = sp.T
    rpt_out[...] = rp.T


def _masked_sims(emb_F, emb_S, edge_mask_u8, bq=256, bn=2048):
    q = edge_mask_u8.shape[0]
    n, d = emb_F.shape
    grid = (q // bq, n // bn)

    qfn = jnp.linalg.norm(emb_F[:q], axis=1, keepdims=True)
    kfn = jnp.linalg.norm(emb_F, axis=1, keepdims=True).reshape(1, n)
    qsn = jnp.linalg.norm(emb_S[:q], axis=1, keepdims=True)
    ksn = jnp.linalg.norm(emb_S, axis=1, keepdims=True).reshape(1, n)

    return pl.pallas_call(
        _sim_mask_body,
        grid=grid,
        in_specs=[
            pl.BlockSpec((bq, d), lambda i, j: (i, 0)),
            pl.BlockSpec((bn, d), lambda i, j: (j, 0)),
            pl.BlockSpec((bq, d), lambda i, j: (i, 0)),
            pl.BlockSpec((bn, d), lambda i, j: (j, 0)),
            pl.BlockSpec((bq, 1), lambda i, j: (i, 0)),
            pl.BlockSpec((1, bn), lambda i, j: (0, j)),
            pl.BlockSpec((bq, 1), lambda i, j: (i, 0)),
            pl.BlockSpec((1, bn), lambda i, j: (0, j)),
            pl.BlockSpec((bq, bn), lambda i, j: (i, j)),
        ],
        out_specs=[
            pl.BlockSpec((bq, bn), lambda i, j: (i, j)),
            pl.BlockSpec((bq, bn), lambda i, j: (i, j)),
            pl.BlockSpec((bn // _W, bq), lambda i, j: (j, i)),
            pl.BlockSpec((bn // _W, bq), lambda i, j: (j, i)),
        ],
        out_shape=[
            jax.ShapeDtypeStruct((q, n), jnp.float32),
            jax.ShapeDtypeStruct((q, n), jnp.float32),
            jax.ShapeDtypeStruct((n // _W, q), jnp.float32),
            jax.ShapeDtypeStruct((n // _W, q), jnp.float32),
        ],
    )(emb_F[:q], emb_F, emb_S[:q], emb_S, qfn, kfn, qsn, ksn, edge_mask_u8)


def _windowed_topk(flat, win_idx, k, n, neg=False):
    win_sorted = jnp.sort(win_idx)
    gidx = win_sorted[:, None] * _W + jnp.arange(_W, dtype=jnp.int32)[None, :]
    cand = jnp.take(flat, gidx.reshape(-1), axis=0)
    if neg:
        cand = -cand
    vals, ci = jax.lax.top_k(cand, k)
    flat_idx = gidx.reshape(-1)[ci]
    inds = jnp.stack([flat_idx // n, flat_idx % n], axis=1)
    return vals, inds


def kernel(emb_F, emb_S, edge_index):
    n, d = emb_F.shape
    q = n // 8
    add_num = 1024
    rm_num = 1024

    row = edge_index[0]
    col = edge_index[1]
    flat_key = jnp.where(row < q, row * n + col, q * n)
    edge_cnt = jnp.zeros((q * n,), dtype=jnp.float32).at[flat_key].add(
        1.0, mode='drop').reshape(q, n)

    s_arr, r_arr, spt, rpt = _masked_sims(emb_F, emb_S, edge_cnt)

    nw = n // _W
    _, ps = jax.lax.top_k(spt.reshape(-1), add_num)
    win_s = (ps % q) * nw + (ps // q)
    add_vals, add_inds = _windowed_topk(s_arr.reshape(-1), win_s, add_num, n)

    _, pr = jax.lax.top_k(-rpt.reshape(-1), rm_num)
    win_r = (pr % q) * nw + (pr // q)
    neg_vals, rm_inds = _windowed_topk(
        r_arr.reshape(-1), win_r, rm_num, n, neg=True)
    rm_vals = -neg_vals
    return add_vals, add_inds, rm_vals, rm_inds
